# baseline (speedup 1.0000x reference)
import functools

import jax
import jax.numpy as jnp
from jax import lax
from jax.experimental import pallas as pl
from jax.experimental.pallas import tpu as pltpu
from jax.experimental.pallas import tpu_sc as plsc

D = 128
_NC, _NS = 2, 16
_NW = _NC * _NS
_HI = lax.Precision.HIGHEST
_RREP = 8


def _wid():
    return lax.axis_index("s") * _NC + lax.axis_index("c")


def _sc_mesh():
    return plsc.VectorSubcoreMesh(core_axis_name="c", subcore_axis_name="s",
                                  num_cores=_NC, num_subcores=_NS)


def _entity_gather(entity_table, ids_pad, npad):
    rows_per_w = npad // _NW
    ch = 80
    n_chunks = rows_per_w // ch

    @functools.partial(
        pl.kernel,
        out_type=jax.ShapeDtypeStruct((npad, D), jnp.float32),
        mesh=_sc_mesh(),
        scratch_types=[
            pltpu.VMEM((rows_per_w,), jnp.int32),
            pltpu.VMEM((ch, D), jnp.float32),
            pltpu.VMEM((ch, D), jnp.float32),
            pltpu.SemaphoreType.DMA,
            pltpu.SemaphoreType.DMA,
        ],
    )
    def k(table_hbm, ids_hbm, out_hbm, idx_v, rows0, rows1, sem0, sem1):
        base = _wid() * rows_per_w
        pltpu.sync_copy(ids_hbm.at[pl.ds(base, rows_per_w)], idx_v)
        rows = (rows0, rows1)
        sems = (sem0, sem1)
        ds = []
        for c in range(n_chunks):
            ds.append(pltpu.async_copy(
                table_hbm.at[idx_v.at[pl.ds(c * ch, ch)]],
                rows[c % 2], sems[c % 2]))
            if c >= 1:
                ds[c - 1].wait()
                pltpu.sync_copy(rows[(c - 1) % 2],
                                out_hbm.at[pl.ds(base + (c - 1) * ch, ch)])
        ds[n_chunks - 1].wait()
        pltpu.sync_copy(rows[(n_chunks - 1) % 2],
                        out_hbm.at[pl.ds(base + (n_chunks - 1) * ch, ch)])

    return k(entity_table, ids_pad)


def _node_proj(ent_rows, W_ent, W1, W3):
    npad = ent_rows.shape[0]
    blk = 512
    grid = npad // blk

    def body(er_ref, we_ref, w1_ref, w3_ref, nt_ref, ah_ref, at_ref):
        nt = jnp.dot(er_ref[...], we_ref[...],
                     preferred_element_type=jnp.float32, precision=_HI)
        nt_ref[...] = nt
        ah_ref[...] = jnp.dot(nt, w1_ref[...],
                              preferred_element_type=jnp.float32, precision=_HI)
        at_ref[...] = jnp.dot(nt, w3_ref[...],
                              preferred_element_type=jnp.float32, precision=_HI)

    w_spec = pl.BlockSpec((D, D), lambda i: (0, 0))
    row_spec = pl.BlockSpec((blk, D), lambda i: (i, 0))
    return pl.pallas_call(
        body,
        grid=(grid,),
        in_specs=[row_spec, w_spec, w_spec, w_spec],
        out_specs=[row_spec, row_spec, row_spec],
        out_shape=[jax.ShapeDtypeStruct((npad, D), jnp.float32)] * 3,
    )(ent_rows, W_ent, W1, W3)


def _small_proj(relation_table, W_rel, W2, b_row, question_emb, W_query):
    nrel = relation_table.shape[0]
    nb = question_emb.shape[0]

    def body(rel_ref, wr_ref, w2_ref, b_ref, q_ref, wq_ref, r_ref, qt_ref):
        rt = jnp.dot(rel_ref[...], wr_ref[...],
                     preferred_element_type=jnp.float32, precision=_HI)
        rr = jnp.dot(rt, w2_ref[...],
                     preferred_element_type=jnp.float32,
                     precision=_HI) + b_ref[...]
        for i in range(_RREP):
            r_ref[pl.ds(i * nrel, nrel), :] = rr
        qt_ref[...] = jnp.dot(q_ref[...], wq_ref[...],
                              preferred_element_type=jnp.float32, precision=_HI)

    return pl.pallas_call(
        body,
        out_shape=[jax.ShapeDtypeStruct((_RREP * nrel, D), jnp.float32),
                   jax.ShapeDtypeStruct((nb, D), jnp.float32)],
    )(relation_table, W_rel, W2, b_row, question_emb, W_query)


def _edge_batch_ptr(heads2d, node_ptr):
    nrows = heads2d.shape[0]
    nb = node_ptr.shape[0] - 1

    def body(h_ref, ptr_ref, eb_ref, ep_ref):
        h = h_ref[...]
        raw = jnp.zeros_like(h)
        kio = lax.broadcasted_iota(jnp.int32, (8, 128), 1)
        acc = jnp.zeros((8, 128), jnp.int32)
        for j in range(1, nb + 1):
            m = (h < ptr_ref[j]).astype(jnp.int32)
            raw = raw + (1 - m)
            cnt = jnp.sum(m)
            acc = acc + jnp.where(kio == j, cnt, 0)
        eb_ref[...] = jnp.minimum(raw, nb - 1)
        ep_ref[...] = acc

    return pl.pallas_call(
        body,
        in_specs=[pl.BlockSpec(memory_space=pltpu.VMEM),
                  pl.BlockSpec(memory_space=pltpu.SMEM)],
        out_shape=[jax.ShapeDtypeStruct((nrows, 128), jnp.int32),
                   jax.ShapeDtypeStruct((8, 128), jnp.int32)],
    )(heads2d, node_ptr)


_FAST_C = 1
_PF, _PS = 20, 20


def _edge_assemble(ah, at_, rtab, heads_pad, tails_pad, attr_pad, epad):
    CH = 128
    assert epad == (_PF + _PS) * 16 * 2 * CH
    ef = _PF * 2 * CH
    es = _PS * 2 * CH

    @functools.partial(
        pl.kernel,
        out_type=jax.ShapeDtypeStruct((epad, D), jnp.float32),
        mesh=_sc_mesh(),
        scratch_types=[
            pltpu.VMEM((ef,), jnp.int32),
            pltpu.VMEM((ef,), jnp.int32),
            pltpu.VMEM((ef,), jnp.int32),
            [pltpu.VMEM((CH, D), jnp.float32)] * 2,
            [pltpu.VMEM((CH, D), jnp.float32)] * 2,
            [pltpu.VMEM((CH, D), jnp.float32)] * 2,
            [pltpu.SemaphoreType.DMA] * 2,
            [pltpu.SemaphoreType.DMA] * 2,
            [pltpu.SemaphoreType.DMA] * 2,
            [pltpu.SemaphoreType.DMA] * 2,
        ],
    )
    def k(ah_hbm, at_hbm, r_hbm, h_hbm, t_hbm, a_hbm, out_hbm,
          ih, it, ia, bh, br, bt, sh, sr, st, ss):
        cid = lax.axis_index("c")
        sid = lax.axis_index("s")
        fast = cid == _FAST_C
        base = jnp.where(fast, sid * ef, 16 * ef + sid * es)
        n_pairs = jnp.where(fast, _PF, _PS)

        @pl.when(fast)
        def _():
            pltpu.sync_copy(h_hbm.at[pl.ds(base, ef)], ih)
            pltpu.sync_copy(t_hbm.at[pl.ds(base, ef)], it)
            pltpu.sync_copy(a_hbm.at[pl.ds(base, ef)], ia)

        @pl.when(jnp.logical_not(fast))
        def _():
            pltpu.sync_copy(h_hbm.at[pl.ds(base, es)], ih.at[pl.ds(0, es)])
            pltpu.sync_copy(t_hbm.at[pl.ds(base, es)], it.at[pl.ds(0, es)])
            pltpu.sync_copy(a_hbm.at[pl.ds(base, es)], ia.at[pl.ds(0, es)])

        def gathers(c, s):
            off = c * CH
            pltpu.async_copy(ah_hbm.at[ih.at[pl.ds(off, CH)]], bh[s], sh[s])
            pltpu.async_copy(r_hbm.at[ia.at[pl.ds(off, CH)]], br[s], sr[s])
            pltpu.async_copy(at_hbm.at[it.at[pl.ds(off, CH)]], bt[s], st[s])

        def wait_gathers(s):
            pltpu.make_async_copy(
                ah_hbm.at[ih.at[pl.ds(0, CH)]], bh[s], sh[s]).wait()
            pltpu.make_async_copy(
                r_hbm.at[ia.at[pl.ds(0, CH)]], br[s], sr[s]).wait()
            pltpu.make_async_copy(
                at_hbm.at[it.at[pl.ds(0, CH)]], bt[s], st[s]).wait()

        def wait_scatter(s):
            pltpu.make_async_copy(
                bh[s], out_hbm.at[pl.ds(base, CH)], ss[s]).wait()

        def process(c, s):
            wait_gathers(s)

            def row(rr):
                for j in range(D // 16):
                    sl = pl.ds(j * 16, 16)
                    bh[s][rr, sl] = (bh[s][rr, sl] + br[s][rr, sl]
                                     + bt[s][rr, sl])

            plsc.parallel_loop(0, CH, 1, unroll=4)(row)
            pltpu.async_copy(bh[s], out_hbm.at[pl.ds(base + c * CH, CH)],
                             ss[s])

        gathers(0, 0)

        def pair(c2, carry):
            c = 2 * c2
            pl.when(c2 > 0)(lambda: wait_scatter(1))
            gathers(c + 1, 1)
            process(c, 0)
            wait_scatter(0)
            pl.when(c2 < n_pairs - 1)(lambda: gathers(c + 2, 0))
            process(c + 1, 1)
            return carry

        lax.fori_loop(0, n_pairs, pair, 0)
        wait_scatter(1)

    return k(ah, at_, rtab, heads_pad, tails_pad, attr_pad)


def kernel(edge_index, node_ptr, edge_attr, question_emb, node_global_ids,
           entity_table, relation_table, W_ent, W_rel, W_query, W_edge, b_edge):
    N = node_global_ids.shape[0]
    E = edge_attr.shape[0]
    NREL = relation_table.shape[0]
    NPAD = ((N + 8 * _NW - 1) // (8 * _NW)) * (8 * _NW)
    EPAD = ((E + 256 * _NW - 1) // (256 * _NW)) * (256 * _NW)

    heads = edge_index[0]
    tails = edge_index[1]
    ids_pad = jnp.concatenate(
        [node_global_ids, jnp.zeros((NPAD - N,), jnp.int32)])
    heads_pad = jnp.concatenate([heads, jnp.full((EPAD - E,), N, jnp.int32)])
    tails_pad = jnp.concatenate([tails, jnp.zeros((EPAD - E,), jnp.int32)])
    attr_pad = jnp.concatenate([edge_attr, jnp.zeros((EPAD - E,), jnp.int32)])
    attr_pad = attr_pad + NREL * (jnp.arange(EPAD, dtype=jnp.int32) % _RREP)
    W1 = W_edge[0:D]
    W2 = W_edge[D:2 * D]
    W3 = W_edge[2 * D:3 * D]

    ent_rows = _entity_gather(entity_table, ids_pad, NPAD)
    nt_pad, ah, at_ = _node_proj(ent_rows, W_ent, W1, W3)
    rtab, question_tokens = _small_proj(
        relation_table, W_rel, W2, b_edge.reshape(1, D), question_emb, W_query)
    eb2d, ep_row = _edge_batch_ptr(heads_pad.reshape(EPAD // 128, 128), node_ptr)
    et_pad = _edge_assemble(ah, at_, rtab, heads_pad, tails_pad, attr_pad, EPAD)

    edge_tokens = et_pad[:E]
    node_tokens = nt_pad[:N]
    edge_batch = eb2d.reshape(EPAD)[:E]
    edge_ptr = ep_row[0, :node_ptr.shape[0]]
    return edge_tokens, node_tokens, question_tokens, edge_batch, edge_ptr


# speedup vs baseline: 1.3643x; 1.3643x over previous
import functools

import jax
import jax.numpy as jnp
from jax import lax
from jax.experimental import pallas as pl
from jax.experimental.pallas import tpu as pltpu
from jax.experimental.pallas import tpu_sc as plsc

D = 128
_NC, _NS = 2, 16
_NW = _NC * _NS
_HI = lax.Precision.DEFAULT
_RREP = 8


def _wid():
    return lax.axis_index("s") * _NC + lax.axis_index("c")


def _sc_mesh():
    return plsc.VectorSubcoreMesh(core_axis_name="c", subcore_axis_name="s",
                                  num_cores=_NC, num_subcores=_NS)


def _entity_gather(entity_table, ids_pad, npad):
    rows_per_w = npad // _NW
    ch = 80
    n_chunks = rows_per_w // ch

    @functools.partial(
        pl.kernel,
        out_type=jax.ShapeDtypeStruct((npad, D), jnp.float32),
        mesh=_sc_mesh(),
        scratch_types=[
            pltpu.VMEM((rows_per_w,), jnp.int32),
            pltpu.VMEM((ch, D), jnp.float32),
            pltpu.VMEM((ch, D), jnp.float32),
            pltpu.SemaphoreType.DMA,
            pltpu.SemaphoreType.DMA,
        ],
    )
    def k(table_hbm, ids_hbm, out_hbm, idx_v, rows0, rows1, sem0, sem1):
        base = _wid() * rows_per_w
        pltpu.sync_copy(ids_hbm.at[pl.ds(base, rows_per_w)], idx_v)
        rows = (rows0, rows1)
        sems = (sem0, sem1)
        ds = []
        for c in range(n_chunks):
            ds.append(pltpu.async_copy(
                table_hbm.at[idx_v.at[pl.ds(c * ch, ch)]],
                rows[c % 2], sems[c % 2]))
            if c >= 1:
                ds[c - 1].wait()
                pltpu.sync_copy(rows[(c - 1) % 2],
                                out_hbm.at[pl.ds(base + (c - 1) * ch, ch)])
        ds[n_chunks - 1].wait()
        pltpu.sync_copy(rows[(n_chunks - 1) % 2],
                        out_hbm.at[pl.ds(base + (n_chunks - 1) * ch, ch)])

    return k(entity_table, ids_pad)


def _node_proj(ent_rows, W_ent, W1, W3):
    npad = ent_rows.shape[0]
    blk = 512
    grid = npad // blk

    def body(er_ref, we_ref, w1_ref, w3_ref, nt_ref, ah_ref, at_ref):
        nt = jnp.dot(er_ref[...], we_ref[...],
                     preferred_element_type=jnp.float32, precision=_HI)
        nt_ref[...] = nt
        ah_ref[...] = jnp.dot(nt, w1_ref[...],
                              preferred_element_type=jnp.float32, precision=_HI)
        at_ref[...] = jnp.dot(nt, w3_ref[...],
                              preferred_element_type=jnp.float32, precision=_HI)

    w_spec = pl.BlockSpec((D, D), lambda i: (0, 0))
    row_spec = pl.BlockSpec((blk, D), lambda i: (i, 0))
    return pl.pallas_call(
        body,
        grid=(grid,),
        in_specs=[row_spec, w_spec, w_spec, w_spec],
        out_specs=[row_spec, row_spec, row_spec],
        out_shape=[jax.ShapeDtypeStruct((npad, D), jnp.float32)] * 3,
    )(ent_rows, W_ent, W1, W3)


def _small_proj(relation_table, W_rel, W2, b_row, question_emb, W_query):
    nrel = relation_table.shape[0]
    nb = question_emb.shape[0]

    def body(rel_ref, wr_ref, w2_ref, b_ref, q_ref, wq_ref, r_ref, qt_ref):
        rt = jnp.dot(rel_ref[...], wr_ref[...],
                     preferred_element_type=jnp.float32, precision=_HI)
        rr = jnp.dot(rt, w2_ref[...],
                     preferred_element_type=jnp.float32,
                     precision=_HI) + b_ref[...]
        for i in range(_RREP):
            r_ref[pl.ds(i * nrel, nrel), :] = rr
        qt_ref[...] = jnp.dot(q_ref[...], wq_ref[...],
                              preferred_element_type=jnp.float32, precision=_HI)

    return pl.pallas_call(
        body,
        out_shape=[jax.ShapeDtypeStruct((_RREP * nrel, D), jnp.float32),
                   jax.ShapeDtypeStruct((nb, D), jnp.float32)],
    )(relation_table, W_rel, W2, b_row, question_emb, W_query)


def _edge_batch_ptr(heads2d, node_ptr):
    nrows = heads2d.shape[0]
    nb = node_ptr.shape[0] - 1

    def body(h_ref, ptr_ref, eb_ref, ep_ref):
        h = h_ref[...]
        raw = jnp.zeros_like(h)
        kio = lax.broadcasted_iota(jnp.int32, (8, 128), 1)
        acc = jnp.zeros((8, 128), jnp.int32)
        for j in range(1, nb + 1):
            m = (h < ptr_ref[j]).astype(jnp.int32)
            raw = raw + (1 - m)
            cnt = jnp.sum(m)
            acc = acc + jnp.where(kio == j, cnt, 0)
        eb_ref[...] = jnp.minimum(raw, nb - 1)
        ep_ref[...] = acc

    return pl.pallas_call(
        body,
        in_specs=[pl.BlockSpec(memory_space=pltpu.VMEM),
                  pl.BlockSpec(memory_space=pltpu.SMEM)],
        out_shape=[jax.ShapeDtypeStruct((nrows, 128), jnp.int32),
                   jax.ShapeDtypeStruct((8, 128), jnp.int32)],
    )(heads2d, node_ptr)


_FAST_C = 1
_PF, _PS = 20, 20


def _edge_assemble(ah, at_, rtab, heads_pad, tails_pad, attr_pad, epad, e):
    CH = 128
    assert epad == (_PF + _PS) * 16 * 2 * CH
    assert e % CH == 0
    ef = _PF * 2 * CH
    es = _PS * 2 * CH

    @functools.partial(
        pl.kernel,
        out_type=[jax.ShapeDtypeStruct((e, D), jnp.float32),
                  jax.ShapeDtypeStruct((epad - e, D), jnp.float32)],
        mesh=_sc_mesh(),
        scratch_types=[
            pltpu.VMEM((ef,), jnp.int32),
            pltpu.VMEM((ef,), jnp.int32),
            pltpu.VMEM((ef,), jnp.int32),
            [pltpu.VMEM((CH, D), jnp.float32)] * 2,
            [pltpu.VMEM((CH, D), jnp.float32)] * 2,
            [pltpu.VMEM((CH, D), jnp.float32)] * 2,
            [pltpu.SemaphoreType.DMA] * 2,
            [pltpu.SemaphoreType.DMA] * 2,
            [pltpu.SemaphoreType.DMA] * 2,
            [pltpu.SemaphoreType.DMA] * 2,
        ],
    )
    def k(ah_hbm, at_hbm, r_hbm, h_hbm, t_hbm, a_hbm, out_hbm, spill_hbm,
          ih, it, ia, bh, br, bt, sh, sr, st, ss):
        cid = lax.axis_index("c")
        sid = lax.axis_index("s")
        fast = cid == _FAST_C
        base = jnp.where(fast, sid * ef, 16 * ef + sid * es)
        n_pairs = jnp.where(fast, _PF, _PS)

        @pl.when(fast)
        def _():
            pltpu.sync_copy(h_hbm.at[pl.ds(base, ef)], ih)
            pltpu.sync_copy(t_hbm.at[pl.ds(base, ef)], it)
            pltpu.sync_copy(a_hbm.at[pl.ds(base, ef)], ia)

        @pl.when(jnp.logical_not(fast))
        def _():
            pltpu.sync_copy(h_hbm.at[pl.ds(base, es)], ih.at[pl.ds(0, es)])
            pltpu.sync_copy(t_hbm.at[pl.ds(base, es)], it.at[pl.ds(0, es)])
            pltpu.sync_copy(a_hbm.at[pl.ds(base, es)], ia.at[pl.ds(0, es)])

        def gathers(c, s):
            off = c * CH
            pltpu.async_copy(ah_hbm.at[ih.at[pl.ds(off, CH)]], bh[s], sh[s])
            pltpu.async_copy(r_hbm.at[ia.at[pl.ds(off, CH)]], br[s], sr[s])
            pltpu.async_copy(at_hbm.at[it.at[pl.ds(off, CH)]], bt[s], st[s])

        def wait_gathers(s):
            pltpu.make_async_copy(
                ah_hbm.at[ih.at[pl.ds(0, CH)]], bh[s], sh[s]).wait()
            pltpu.make_async_copy(
                r_hbm.at[ia.at[pl.ds(0, CH)]], br[s], sr[s]).wait()
            pltpu.make_async_copy(
                at_hbm.at[it.at[pl.ds(0, CH)]], bt[s], st[s]).wait()

        def wait_scatter(s):
            pltpu.make_async_copy(
                bh[s], out_hbm.at[pl.ds(0, CH)], ss[s]).wait()

        def process(c, s):
            wait_gathers(s)

            def row(rr):
                for j in range(D // 16):
                    sl = pl.ds(j * 16, 16)
                    bh[s][rr, sl] = (bh[s][rr, sl] + br[s][rr, sl]
                                     + bt[s][rr, sl])

            plsc.parallel_loop(0, CH, 1, unroll=4)(row)
            tgt = base + c * CH

            @pl.when(tgt < e)
            def _():
                pltpu.async_copy(bh[s], out_hbm.at[pl.ds(tgt, CH)], ss[s])

            @pl.when(tgt >= e)
            def _():
                pltpu.async_copy(bh[s], spill_hbm.at[pl.ds(tgt - e, CH)],
                                 ss[s])

        gathers(0, 0)

        def pair(c2, carry):
            c = 2 * c2
            pl.when(c2 > 0)(lambda: wait_scatter(1))
            gathers(c + 1, 1)
            process(c, 0)
            wait_scatter(0)
            pl.when(c2 < n_pairs - 1)(lambda: gathers(c + 2, 0))
            process(c + 1, 1)
            return carry

        lax.fori_loop(0, n_pairs, pair, 0)
        wait_scatter(1)

    return k(ah, at_, rtab, heads_pad, tails_pad, attr_pad)


def kernel(edge_index, node_ptr, edge_attr, question_emb, node_global_ids,
           entity_table, relation_table, W_ent, W_rel, W_query, W_edge, b_edge):
    N = node_global_ids.shape[0]
    E = edge_attr.shape[0]
    NREL = relation_table.shape[0]
    NPAD = ((N + 8 * _NW - 1) // (8 * _NW)) * (8 * _NW)
    EPAD = ((E + 256 * _NW - 1) // (256 * _NW)) * (256 * _NW)

    heads = edge_index[0]
    tails = edge_index[1]
    ids_pad = jnp.concatenate(
        [node_global_ids, jnp.zeros((NPAD - N,), jnp.int32)])
    heads_pad = jnp.concatenate([heads, jnp.full((EPAD - E,), N, jnp.int32)])
    tails_pad = jnp.concatenate([tails, jnp.zeros((EPAD - E,), jnp.int32)])
    attr_pad = jnp.concatenate([edge_attr, jnp.zeros((EPAD - E,), jnp.int32)])
    attr_pad = attr_pad + NREL * (jnp.arange(EPAD, dtype=jnp.int32) % _RREP)
    W1 = W_edge[0:D]
    W2 = W_edge[D:2 * D]
    W3 = W_edge[2 * D:3 * D]

    ent_rows = _entity_gather(entity_table, ids_pad, NPAD)
    nt_pad, ah, at_ = _node_proj(ent_rows, W_ent, W1, W3)
    rtab, question_tokens = _small_proj(
        relation_table, W_rel, W2, b_edge.reshape(1, D), question_emb, W_query)
    eb2d, ep_row = _edge_batch_ptr(heads_pad.reshape(EPAD // 128, 128), node_ptr)
    et_main, _et_spill = _edge_assemble(
        ah, at_, rtab, heads_pad, tails_pad, attr_pad, EPAD, E)

    edge_tokens = et_main
    node_tokens = nt_pad[:N]
    edge_batch = eb2d.reshape(EPAD)[:E]
    edge_ptr = ep_row[0, :node_ptr.shape[0]]
    return edge_tokens, node_tokens, question_tokens, edge_batch, edge_ptr
